# baseline (speedup 1.0000x reference)
import functools

import jax
import jax.numpy as jnp
from jax import lax
from jax.experimental import pallas as pl
from jax.experimental.pallas import tpu as pltpu
from jax.experimental.pallas import tpu_sc as plsc

_RBLK = 16


def _sc_gather_pairs(table2, idx2):
    info = plsc.get_sparse_core_info()
    nc, ns = info.num_cores, info.num_subcores
    nw = nc * ns
    b, d2 = idx2.shape[0], table2.shape[1]
    b_per_w = b // nw
    mesh = plsc.VectorSubcoreMesh(core_axis_name="c", subcore_axis_name="s")

    @functools.partial(
        pl.kernel,
        mesh=mesh,
        out_type=jax.ShapeDtypeStruct((b, d2), jnp.float32),
        scratch_types=[
            pltpu.VMEM((b_per_w,), jnp.int32),
            pltpu.VMEM((b_per_w, d2), jnp.float32),
            pltpu.SemaphoreType.DMA,
        ],
    )
    def gather_kernel(table_hbm, idx_hbm, out_hbm, idx_v, rows_v, sem):
        wid = lax.axis_index("s") * nc + lax.axis_index("c")
        base = wid * b_per_w
        pltpu.sync_copy(idx_hbm.at[pl.ds(base, b_per_w)], idx_v)
        pltpu.async_copy(table_hbm.at[idx_v], rows_v, sem).wait()
        pltpu.sync_copy(rows_v, out_hbm.at[pl.ds(base, b_per_w)])

    return gather_kernel(table2, idx2)


def _mm_body(pairs_ref, par_ref, b_ref, w_hbm, o_ref, w_scr, sem):
    d = w_scr.shape[1]

    @pl.when(pl.program_id(0) == 0)
    def _():
        pltpu.make_async_copy(w_hbm, w_scr, sem).start()
        pltpu.make_async_copy(w_hbm, w_scr, sem).wait()

    e = jnp.where(par_ref[...] == 1, pairs_ref[:, d:], pairs_ref[:, :d])
    o_ref[...] = (
        lax.dot_general(
            e,
            w_scr[...],
            (((1,), (1,)), ((), ())),
            preferred_element_type=jnp.float32,
        )
        + b_ref[...]
    )


def _tc_project(pairs, parity, W, b):
    bsz, d2 = pairs.shape
    d = d2 // 2
    vocab = W.shape[0]
    nr = bsz // _RBLK
    return pl.pallas_call(
        _mm_body,
        grid=(nr,),
        in_specs=[
            pl.BlockSpec((_RBLK, d2), lambda i: (i, 0)),
            pl.BlockSpec((_RBLK, 1), lambda i: (i, 0)),
            pl.BlockSpec((1, vocab), lambda i: (0, 0)),
            pl.BlockSpec(memory_space=pl.ANY),
        ],
        out_specs=pl.BlockSpec((_RBLK, vocab), lambda i: (i, 0)),
        out_shape=jax.ShapeDtypeStruct((bsz, vocab), jnp.float32),
        compiler_params=pltpu.CompilerParams(
            vmem_limit_bytes=63 * 1024 * 1024,
        ),
        scratch_shapes=[
            pltpu.VMEM((vocab, d), jnp.float32),
            pltpu.SemaphoreType.DMA,
        ],
    )(pairs, parity, b.reshape(1, vocab), W)


def kernel(inputs, table, W, b):
    vocab, d = table.shape
    table2 = table.reshape(vocab // 2, 2 * d)
    pairs = _sc_gather_pairs(table2, inputs >> 1)
    parity = (inputs & 1).reshape(inputs.shape[0], 1)
    return _tc_project(pairs, parity, W, b)


# speedup vs baseline: 1.5912x; 1.5912x over previous
import jax
import jax.numpy as jnp
from jax import lax
from jax.experimental import pallas as pl
from jax.experimental.pallas import tpu as pltpu

_VBLK = 2048


def _gather_body(idx_ref, table_hbm, o_ref, sem):
    n = o_ref.shape[0]

    def issue(i, carry):
        pltpu.make_async_copy(
            table_hbm.at[pl.ds(idx_ref[i], 1)], o_ref.at[pl.ds(i, 1)], sem
        ).start()
        return carry

    lax.fori_loop(0, n, issue, 0)

    def drain(i, carry):
        pltpu.make_async_copy(
            table_hbm.at[pl.ds(idx_ref[i], 1)], o_ref.at[pl.ds(i, 1)], sem
        ).wait()
        return carry

    lax.fori_loop(0, n, drain, 0)


def _tc_gather(table, inputs):
    bsz = inputs.shape[0]
    d = table.shape[1]
    return pl.pallas_call(
        _gather_body,
        in_specs=[
            pl.BlockSpec(memory_space=pltpu.SMEM),
            pl.BlockSpec(memory_space=pl.ANY),
        ],
        out_specs=pl.BlockSpec((bsz, d), lambda: (0, 0)),
        out_shape=jax.ShapeDtypeStruct((bsz, d), jnp.float32),
        scratch_shapes=[pltpu.SemaphoreType.DMA],
    )(inputs, table)


def _mm_body(e_ref, w_ref, b_ref, o_ref):
    o_ref[...] = (
        lax.dot_general(
            e_ref[...],
            w_ref[...],
            (((1,), (1,)), ((), ())),
            preferred_element_type=jnp.float32,
        )
        + b_ref[...]
    )


def _tc_project(embeds, W, b):
    bsz, d = embeds.shape
    vocab = W.shape[0]
    nv = pl.cdiv(vocab, _VBLK)
    return pl.pallas_call(
        _mm_body,
        grid=(nv,),
        in_specs=[
            pl.BlockSpec((bsz, d), lambda i: (0, 0)),
            pl.BlockSpec((_VBLK, d), lambda i: (i, 0)),
            pl.BlockSpec((1, _VBLK), lambda i: (0, i)),
        ],
        out_specs=pl.BlockSpec((bsz, _VBLK), lambda i: (0, i)),
        out_shape=jax.ShapeDtypeStruct((bsz, vocab), jnp.float32),
        compiler_params=pltpu.CompilerParams(
            dimension_semantics=("parallel",),
        ),
    )(embeds, W, b.reshape(1, vocab))


def kernel(inputs, table, W, b):
    embeds = _tc_gather(table, inputs)
    return _tc_project(embeds, W, b)


# speedup vs baseline: 3.7795x; 2.3752x over previous
import jax
import jax.numpy as jnp
from jax import lax
from jax.experimental import pallas as pl
from jax.experimental.pallas import tpu as pltpu

_VBLK = 2048


def _gather_body(idx_ref, table_hbm, o_ref, sem):
    n = o_ref.shape[0]

    def issue(i, carry):
        pltpu.make_async_copy(
            table_hbm.at[pl.ds(idx_ref[i], 1)], o_ref.at[pl.ds(i, 1)], sem
        ).start()
        return carry

    lax.fori_loop(0, n, issue, 0)

    def drain(i, carry):
        pltpu.make_async_copy(
            table_hbm.at[pl.ds(idx_ref[i], 1)], o_ref.at[pl.ds(i, 1)], sem
        ).wait()
        return carry

    lax.fori_loop(0, n, drain, 0)


def _tc_gather(table, inputs):
    bsz = inputs.shape[0]
    d = table.shape[1]
    return pl.pallas_call(
        _gather_body,
        in_specs=[
            pl.BlockSpec(memory_space=pltpu.SMEM),
            pl.BlockSpec(memory_space=pl.ANY),
        ],
        out_specs=pl.BlockSpec((bsz, d), lambda: (0, 0)),
        out_shape=jax.ShapeDtypeStruct((bsz, d), jnp.float32),
        scratch_shapes=[pltpu.SemaphoreType.DMA],
    )(inputs, table)


def _mm_body(e_ref, wt_ref, b_ref, o_ref):
    o_ref[...] = (
        lax.dot_general(
            wt_ref[...],
            e_ref[...],
            (((0,), (1,)), ((), ())),
            preferred_element_type=jnp.float32,
        )
        + b_ref[...]
    )


def _tc_project_t(embeds, WT, b2):
    bsz, d = embeds.shape
    vocab = WT.shape[1]
    nv = pl.cdiv(vocab, _VBLK)
    return pl.pallas_call(
        _mm_body,
        grid=(nv,),
        in_specs=[
            pl.BlockSpec((bsz, d), lambda i: (0, 0)),
            pl.BlockSpec((d, _VBLK), lambda i: (0, i)),
            pl.BlockSpec((_VBLK, 1), lambda i: (i, 0)),
        ],
        out_specs=pl.BlockSpec((_VBLK, bsz), lambda i: (i, 0)),
        out_shape=jax.ShapeDtypeStruct((vocab, bsz), jnp.float32),
        compiler_params=pltpu.CompilerParams(
            dimension_semantics=("parallel",),
        ),
    )(embeds, WT, b2)


def kernel(inputs, table, W, b):
    vocab = W.shape[0]
    embeds = _tc_gather(table, inputs)
    out_t = _tc_project_t(embeds, W.T, b.reshape(vocab, 1))
    return out_t.T


# speedup vs baseline: 3.8436x; 1.0170x over previous
import jax
import jax.numpy as jnp
from jax import lax
from jax.experimental import pallas as pl
from jax.experimental.pallas import tpu as pltpu

_VBLK = 4096


def _gather_body(idx_ref, table_hbm, o_ref, sem):
    n = o_ref.shape[0]

    def issue(i, carry):
        pltpu.make_async_copy(
            table_hbm.at[pl.ds(idx_ref[i], 1)], o_ref.at[pl.ds(i, 1)], sem
        ).start()
        return carry

    lax.fori_loop(0, n, issue, 0)

    def drain(i, carry):
        pltpu.make_async_copy(
            table_hbm.at[pl.ds(idx_ref[i], 1)], o_ref.at[pl.ds(i, 1)], sem
        ).wait()
        return carry

    lax.fori_loop(0, n, drain, 0)


def _tc_gather(table, inputs):
    bsz = inputs.shape[0]
    d = table.shape[1]
    return pl.pallas_call(
        _gather_body,
        in_specs=[
            pl.BlockSpec(memory_space=pltpu.SMEM),
            pl.BlockSpec(memory_space=pl.ANY),
        ],
        out_specs=pl.BlockSpec((bsz, d), lambda: (0, 0)),
        out_shape=jax.ShapeDtypeStruct((bsz, d), jnp.float32),
        scratch_shapes=[pltpu.SemaphoreType.DMA],
    )(inputs, table)


def _mm_body(e_ref, wt_ref, b_ref, o_ref):
    o_ref[...] = (
        lax.dot_general(
            wt_ref[...],
            e_ref[...],
            (((0,), (1,)), ((), ())),
            preferred_element_type=jnp.float32,
        )
        + b_ref[...]
    )


def _tc_project_t(embeds, WT, b2):
    bsz, d = embeds.shape
    vocab = WT.shape[1]
    nv = pl.cdiv(vocab, _VBLK)
    return pl.pallas_call(
        _mm_body,
        grid=(nv,),
        in_specs=[
            pl.BlockSpec((bsz, d), lambda i: (0, 0)),
            pl.BlockSpec((d, _VBLK), lambda i: (0, i)),
            pl.BlockSpec((_VBLK, 1), lambda i: (i, 0)),
        ],
        out_specs=pl.BlockSpec((_VBLK, bsz), lambda i: (i, 0)),
        out_shape=jax.ShapeDtypeStruct((vocab, bsz), jnp.float32),
        compiler_params=pltpu.CompilerParams(
            dimension_semantics=("parallel",),
        ),
    )(embeds, WT, b2)


def kernel(inputs, table, W, b):
    vocab = W.shape[0]
    embeds = _tc_gather(table, inputs)
    out_t = _tc_project_t(embeds, W.T, b.reshape(vocab, 1))
    return out_t.T
